# Initial kernel scaffold; baseline (speedup 1.0000x reference)
import jax
import jax.numpy as jnp
from jax.experimental import pallas as pl


def kernel(x, edge_attr, params, edge_index, batch):
    raise NotImplementedError("write your pallas kernel here")


# speedup vs baseline: 3.5336x; 3.5336x over previous
import functools

import jax
import jax.numpy as jnp
from jax import lax
from jax.experimental import pallas as pl
from jax.experimental.pallas import tpu as pltpu
from jax.experimental.pallas import tpu_sc as plsc

_N = 10000
_E = 320000
_H = 128
_NG = 64
_MD = 16

_W = 128
_NROW = _E // _W
_NWORK = 32
_STRIPE = 624
_SCH = 104
_TAIL = _N - 16 * _STRIPE


def _matmul_bias_body(x_ref, w_ref, b_ref, o_ref):
    o_ref[...] = (
        jnp.dot(x_ref[...], w_ref[...], preferred_element_type=jnp.float32)
        + b_ref[...]
    )


def _node_emb(x, w, b):
    return pl.pallas_call(
        _matmul_bias_body,
        out_shape=jax.ShapeDtypeStruct((_N, _H), jnp.float32),
    )(x, w, b.reshape(1, _H))


def _edge_emb(ea, w, b):
    blk = 8000
    grid = (_E // blk,)
    return pl.pallas_call(
        _matmul_bias_body,
        grid=grid,
        in_specs=[
            pl.BlockSpec((blk, ea.shape[1]), lambda i: (i, 0)),
            pl.BlockSpec((ea.shape[1], _H), lambda i: (0, 0)),
            pl.BlockSpec((1, _H), lambda i: (0, 0)),
        ],
        out_specs=pl.BlockSpec((blk, _H), lambda i: (i, 0)),
        out_shape=jax.ShapeDtypeStruct((_E, _H), jnp.float32),
    )(ea, w, b.reshape(1, _H))


def _mlp_bn_body(h_ref, agg_ref, w1_ref, b1_ref, w2_ref, b2_ref, g_ref,
                 bb_ref, o_ref):
    z = h_ref[...] + agg_ref[0] + agg_ref[1]
    z1 = jnp.maximum(
        jnp.dot(z, w1_ref[...], preferred_element_type=jnp.float32)
        + b1_ref[...], 0.0)
    z2 = (jnp.dot(z1, w2_ref[...], preferred_element_type=jnp.float32)
          + b2_ref[...])
    mean = jnp.mean(z2, axis=0, keepdims=True)
    var = jnp.mean((z2 - mean) ** 2, axis=0, keepdims=True)
    zn = (z2 - mean) * lax.rsqrt(var + 1e-5) * g_ref[...] + bb_ref[...]
    o_ref[...] = jnp.maximum(zn, 0.0)


def _mlp_bn(h, agg, lp):
    return pl.pallas_call(
        _mlp_bn_body,
        out_shape=jax.ShapeDtypeStruct((_N, _H), jnp.float32),
    )(h, agg, lp["W1"], lp["b1"].reshape(1, -1), lp["W2"],
      lp["b2"].reshape(1, -1), lp["bn_g"].reshape(1, -1),
      lp["bn_b"].reshape(1, -1))


def _pool_head_body(h_ref, b_ref, wc_ref, bc_ref, wu_ref, bu_ref, wfu_ref,
                    wfc_ref, bf_ref, o_ref):
    gids = lax.broadcasted_iota(jnp.int32, (_N, _NG), 1)
    onehot = (b_ref[...] == gids).astype(jnp.float32)
    sums = lax.dot_general(onehot, h_ref[...], (((0,), (0,)), ((), ())),
                           preferred_element_type=jnp.float32)
    counts = jnp.sum(onehot, axis=0)[:, None]
    gx = sums / jnp.maximum(counts, 1.0)
    eu = jnp.dot(gx, wu_ref[...], preferred_element_type=jnp.float32) + bu_ref[...]
    ec = jnp.dot(gx, wc_ref[...], preferred_element_type=jnp.float32) + bc_ref[...]
    o_ref[...] = (
        jnp.dot(eu, wfu_ref[...], preferred_element_type=jnp.float32)
        + jnp.dot(ec, wfc_ref[...], preferred_element_type=jnp.float32)
        + bf_ref[...])


def _pool_head(h, batch, params):
    wf = params["final"]["W"]
    nc = wf.shape[1]
    return pl.pallas_call(
        _pool_head_body,
        out_shape=jax.ShapeDtypeStruct((_NG, nc), jnp.float32),
    )(h, batch.reshape(_N, 1),
      params["lin_common"]["W"], params["lin_common"]["b"].reshape(1, -1),
      params["lin_uncommon"]["W"], params["lin_uncommon"]["b"].reshape(1, -1),
      wf[:_MD], wf[_MD:], params["final"]["b"].reshape(1, -1))


def _sc_message(h, e, src2, dst2):
    mesh = plsc.VectorSubcoreMesh(core_axis_name="c", subcore_axis_name="s")

    @functools.partial(
        pl.kernel,
        mesh=mesh,
        out_type=jax.ShapeDtypeStruct((2, _N, _H), jnp.float32),
        scratch_types=[
            pltpu.VMEM((1, _W), jnp.int32),
            pltpu.VMEM((1, _W), jnp.int32),
            pltpu.VMEM((_W, _H), jnp.float32),
            pltpu.VMEM((_W, _H), jnp.float32),
            pltpu.VMEM_SHARED((_N, _H), jnp.float32),
            pltpu.SemaphoreType.DMA,
        ],
    )
    def k(h_hbm, e_hbm, src_hbm, dst_hbm, out_hbm, src_v, dst_v, rows_v,
          e_v, agg_sh, sem):
        c = lax.axis_index("c")
        s = lax.axis_index("s")
        wid = c * 16 + s

        zero16 = jnp.zeros((16,), jnp.float32)

        def zrow(r, carry):
            for j in range(8):
                rows_v[r, pl.ds(j * 16, 16)] = zero16
            return carry

        lax.fori_loop(0, _W, zrow, 0)
        r0 = s * _STRIPE
        for t in range(_STRIPE // _SCH):
            pltpu.sync_copy(rows_v.at[pl.ds(0, _SCH)],
                            agg_sh.at[pl.ds(r0 + t * _SCH, _SCH)])

        @pl.when(s == 15)
        def _zero_tail():
            pltpu.sync_copy(rows_v.at[pl.ds(0, _TAIL)],
                            agg_sh.at[pl.ds(16 * _STRIPE, _TAIL)])

        plsc.subcore_barrier()

        lo = (wid * _NROW) // _NWORK
        hi = ((wid + 1) * _NROW) // _NWORK

        def chunk(r, carry):
            pltpu.sync_copy(src_hbm.at[pl.ds(r, 1)], src_v)
            pltpu.sync_copy(dst_hbm.at[pl.ds(r, 1)], dst_v)
            cp = pltpu.async_copy(h_hbm.at[src_v.at[0]], rows_v, sem)
            pltpu.sync_copy(e_hbm.at[pl.ds(r * _W, _W)], e_v)
            cp.wait()

            def crow(rr, cy):
                for j in range(8):
                    sl = pl.ds(j * 16, 16)
                    rows_v[rr, sl] = jnp.maximum(
                        rows_v[rr, sl] + e_v[rr, sl], 0.0)
                return cy

            lax.fori_loop(0, _W, crow, 0)
            pltpu.sync_copy(rows_v, agg_sh.at[dst_v.at[0]], add=True)
            return carry

        lax.fori_loop(lo, hi, chunk, 0)
        plsc.subcore_barrier()

        for t in range(_STRIPE // _SCH):
            rr = r0 + t * _SCH
            pltpu.sync_copy(agg_sh.at[pl.ds(rr, _SCH)],
                            rows_v.at[pl.ds(0, _SCH)])
            pltpu.sync_copy(rows_v.at[pl.ds(0, _SCH)],
                            out_hbm.at[c, pl.ds(rr, _SCH)])

        @pl.when(s == 15)
        def _write_tail():
            pltpu.sync_copy(agg_sh.at[pl.ds(16 * _STRIPE, _TAIL)],
                            rows_v.at[pl.ds(0, _TAIL)])
            pltpu.sync_copy(rows_v.at[pl.ds(0, _TAIL)],
                            out_hbm.at[c, pl.ds(16 * _STRIPE, _TAIL)])

    return k(h, e, src2, dst2)


def kernel(x, edge_attr, params, edge_index, batch):
    src2 = edge_index[0].reshape(_NROW, _W)
    dst2 = edge_index[1].reshape(_NROW, _W)
    h = _node_emb(x, params["node_emb"]["W"], params["node_emb"]["b"])
    e = _edge_emb(edge_attr, params["edge_emb"]["W"], params["edge_emb"]["b"])
    for lp in params["layers"]:
        agg = _sc_message(h, e, src2, dst2)
        h = _mlp_bn(h, agg, lp)
    return _pool_head(h, batch, params)
